# baseline (speedup 1.0000x reference)
import functools

import jax
import jax.numpy as jnp
from jax import lax
from jax.experimental import pallas as pl
from jax.experimental.pallas import tpu as pltpu
from jax.experimental.pallas import tpu_sc as plsc

F = 2
B = 16384
H = 20
D = 32
V = 1000000
N = B * H
NC = 2
NS = 16
NW = NC * NS
PER_W = N // NW
CH = 1024
NCH = PER_W // CH
NBUF = 3

_mesh = plsc.VectorSubcoreMesh(core_axis_name="c", subcore_axis_name="s")


@functools.partial(
    pl.kernel,
    mesh=_mesh,
    compiler_params=pltpu.CompilerParams(use_tc_tiling_on_sc=False),
    out_type=jax.ShapeDtypeStruct((N, D), jnp.float32),
    scratch_types=[
        pltpu.VMEM((NBUF, CH), jnp.int32),
        pltpu.VMEM((NBUF, CH, D), jnp.float32),
        pltpu.SemaphoreType.DMA,
        pltpu.SemaphoreType.DMA,
        pltpu.SemaphoreType.DMA,
    ],
)
def _lookup1(idx_hbm, tab_hbm, out_hbm, idx_v, rows_v, sem0, sem1, sem2):
    wid = lax.axis_index("s") * NC + lax.axis_index("c")
    base = wid * PER_W
    sems = (sem0, sem1, sem2)

    def start(j):
        slot = j % NBUF
        off = base + j * CH
        pltpu.sync_copy(idx_hbm.at[pl.ds(off, CH)], idx_v.at[slot])
        return pltpu.async_copy(
            tab_hbm.at[idx_v.at[slot]], rows_v.at[slot], sems[slot])

    inflight = {j: start(j) for j in range(NBUF - 1)}
    for j in range(NCH):
        if j + NBUF - 1 < NCH:
            inflight[j + NBUF - 1] = start(j + NBUF - 1)
        inflight[j].wait()
        slot = j % NBUF
        pltpu.sync_copy(rows_v.at[slot], out_hbm.at[pl.ds(base + j * CH, CH)])


def kernel(indices, table_0, table_1):
    idx = indices.reshape(F, N).astype(jnp.int32)
    o0 = _lookup1(idx[0], table_0)
    o1 = _lookup1(idx[1], table_1)
    return jnp.stack([o0.reshape(B, H, D), o1.reshape(B, H, D)], axis=0)


# speedup vs baseline: 1.0010x; 1.0010x over previous
import functools

import jax
import jax.numpy as jnp
from jax import lax
from jax.experimental import pallas as pl
from jax.experimental.pallas import tpu as pltpu
from jax.experimental.pallas import tpu_sc as plsc

F = 2
B = 16384
H = 20
D = 32
V = 1000000
N = B * H
NC = 2
NS = 16
NW = NC * NS
PER_W = N // NW
CH = 1024
NCH = PER_W // CH
NBUF = 3

_mesh = plsc.VectorSubcoreMesh(core_axis_name="c", subcore_axis_name="s")


@functools.partial(
    pl.kernel,
    mesh=_mesh,
    compiler_params=pltpu.CompilerParams(use_tc_tiling_on_sc=False),
    out_type=jax.ShapeDtypeStruct((N, D), jnp.float32),
    scratch_types=[
        pltpu.VMEM((NBUF, CH), jnp.int32),
        pltpu.VMEM((NBUF, CH, D), jnp.float32),
        pltpu.SemaphoreType.DMA,
        pltpu.SemaphoreType.DMA,
        pltpu.SemaphoreType.DMA,
    ],
)
def _lookup1(idx_hbm, tab_hbm, out_hbm, idx_v, rows_v, sem0, sem1, sem2):
    wid = lax.axis_index("s") * NC + lax.axis_index("c")
    base = wid * PER_W
    sems = (sem0, sem1, sem2)

    def start(j):
        slot = j % NBUF
        off = base + j * CH
        pltpu.sync_copy(idx_hbm.at[pl.ds(off, CH)], idx_v.at[slot])
        return pltpu.async_copy(
            tab_hbm.at[idx_v.at[slot]], rows_v.at[slot], sems[slot])

    inflight = {j: start(j) for j in range(NBUF - 1)}
    for j in range(NCH):
        if j + NBUF - 1 < NCH:
            inflight[j + NBUF - 1] = start(j + NBUF - 1)
        inflight[j].wait()
        slot = j % NBUF
        pltpu.sync_copy(rows_v.at[slot], out_hbm.at[pl.ds(base + j * CH, CH)])


def kernel(indices, table_0, table_1):
    i0 = indices[0].reshape(N).astype(jnp.int32)
    i1 = indices[1].reshape(N).astype(jnp.int32)
    o0 = _lookup1(i0, table_0)
    o1 = _lookup1(i1, table_1)
    return jnp.stack([o0.reshape(B, H, D), o1.reshape(B, H, D)], axis=0)
